# baseline (speedup 1.0000x reference)
import functools

import jax
import jax.numpy as jnp
from jax.experimental import pallas as pl
from jax.experimental.pallas import tpu as pltpu

N_LAYERS = 16
DIM = 64
RES_DIM = 62
N = 256
E_HID = (DIM * 2 + 1) * 2
BI = 32
NBLK = N // BI

_f32 = jnp.float32


def _silu(x):
    return x * jax.nn.sigmoid(x)


def _egnn_kernel(
    seq_ref, chain_ref, tab_ref, coords0_ref, mask_ref,
    w1i_ref, w1j_ref, w1d_ref, eb1_ref,
    ew2_ref, eb2_ref,
    cw1_ref, cb1_ref, cw2_ref, cb2_ref,
    lg_ref, lb_ref, cns_ref,
    nw1_ref, nb1_ref, nw2_ref, nb2_ref,
    feats_out_ref, coors_out_ref,
    feats_s, coors_s, mi_s, cnew_s,
):
    l = pl.program_id(0)

    @pl.when(l == 0)
    def _init():
        v = jax.lax.broadcasted_iota(jnp.int32, (N, 24), 1)
        oh = jnp.logical_or(v == seq_ref[...], v == chain_ref[...] + 21)
        feats_s[...] = jnp.dot(oh.astype(_f32), tab_ref[...],
                               preferred_element_type=_f32)
        coors_s[...] = coords0_ref[...]

    feats = feats_s[...]
    C = coors_s[...]

    w1i = w1i_ref[0]
    w1j = w1j_ref[0]
    w1d = w1d_ref[0]
    eb1 = eb1_ref[0]
    ew2 = ew2_ref[0]
    eb2 = eb2_ref[0]
    cw1 = cw1_ref[0]
    cb1 = cb1_ref[0]
    cw2 = cw2_ref[0]
    cb2 = cb2_ref[0][0, 0]
    cns = cns_ref[0][0, 0]

    A = jnp.dot(feats, w1i, preferred_element_type=_f32) + eb1
    B = jnp.dot(feats, w1j, preferred_element_type=_f32)

    CT = jnp.transpose(C[:, :8])
    dx = C[:, 0:1] - CT[0:1, :]
    dy = C[:, 1:2] - CT[1:2, :]
    dz = C[:, 2:3] - CT[2:3, :]
    D = dx * dx + dy * dy + dz * dz
    norm = jnp.sqrt(jnp.maximum(D, 1e-12))
    rinv = cns / jnp.maximum(norm, 1e-8)
    ii = jax.lax.broadcasted_iota(jnp.int32, (N, N), 0)
    jj = jax.lax.broadcasted_iota(jnp.int32, (N, N), 1)
    rinv = jnp.where(ii == jj, 0.0, rinv)

    for ib in range(NBLK):
        i0 = ib * BI
        A_blk = A[i0:i0 + BI, :]
        D_blk = D[i0:i0 + BI, :]
        rinv_blk = rinv[i0:i0 + BI, :]
        C_blk = C[i0:i0 + BI, :]

        D3 = D_blk[:, :, None]
        H = A_blk[:, None, :] + B[None, :, :] + D3 * w1d[None, :, :]
        H2 = H.reshape(BI * N, E_HID)
        m2 = _silu(jnp.dot(_silu(H2), ew2, preferred_element_type=_f32) + eb2)
        u2 = _silu(jnp.dot(m2, cw1, preferred_element_type=_f32) + cb1)
        w = jnp.sum(u2.reshape(BI, N, N) * cw2[None, :, :], axis=-1) + cb2
        wc = jnp.clip(w, -2.0, 2.0)
        R = wc * rinv_blk

        mi_s[pl.ds(i0, BI), :] = jnp.sum(m2.reshape(BI, N, DIM), axis=1)
        rs = jnp.sum(R, axis=1, keepdims=True)
        delta = C_blk * rs - jnp.dot(R, C, preferred_element_type=_f32)
        cnew_s[pl.ds(i0, BI), :] = C_blk + delta

    mi = mi_s[...]
    mu = jnp.mean(feats, axis=-1, keepdims=True)
    var = jnp.mean((feats - mu) ** 2, axis=-1, keepdims=True)
    normed = (feats - mu) * jax.lax.rsqrt(var + 1e-5) * lg_ref[0] + lb_ref[0]
    h = jnp.concatenate([normed, mi], axis=-1)
    h = _silu(jnp.dot(h, nw1_ref[0], preferred_element_type=_f32) + nb1_ref[0])
    node = jnp.dot(h, nw2_ref[0], preferred_element_type=_f32) + nb2_ref[0] + feats

    maskf = mask_ref[...]
    cupd = maskf * cnew_s[...] + (1.0 - maskf) * coords0_ref[...]

    feats_s[...] = node
    coors_s[...] = cupd
    feats_out_ref[...] = node
    coors_out_ref[...] = cupd


@jax.jit
def kernel(input_seq, input_chain, input_mask, coords, seq_table, chain_table,
           edge_W1, edge_b1, edge_W2, edge_b2, ln_g, ln_b, cn_scale,
           node_W1, node_b1, node_W2, node_b2, coor_W1, coor_b1, coor_W2,
           coor_b2):
    W1i = edge_W1[:, 0:DIM, :]
    W1j = edge_W1[:, DIM:2 * DIM, :]
    w1d = edge_W1[:, 2 * DIM:2 * DIM + 1, :]
    tab = jnp.zeros((24, DIM), _f32)
    tab = tab.at[0:21, 0:RES_DIM].set(seq_table)
    tab = tab.at[21:23, RES_DIM:DIM].set(chain_table)
    seq2 = input_seq.astype(jnp.int32).reshape(N, 1)
    chain2 = input_chain.astype(jnp.int32).reshape(N, 1)
    coords_pad = jnp.concatenate(
        [coords[0].astype(_f32), jnp.zeros((N, 125), _f32)], axis=1)
    maskf = input_mask.astype(_f32).reshape(N, 1)

    per_layer = lambda s: pl.BlockSpec((1,) + s[1:], lambda l: (l, 0, 0))
    full = lambda s: pl.BlockSpec(s, lambda l: (0, 0))

    ins = [
        (seq2, full((N, 1))),
        (chain2, full((N, 1))),
        (tab, full((24, DIM))),
        (coords_pad, full((N, 128))),
        (maskf, full((N, 1))),
        (W1i, per_layer((N_LAYERS, DIM, E_HID))),
        (W1j, per_layer((N_LAYERS, DIM, E_HID))),
        (w1d, per_layer((N_LAYERS, 1, E_HID))),
        (edge_b1[:, None, :], per_layer((N_LAYERS, 1, E_HID))),
        (edge_W2, per_layer((N_LAYERS, E_HID, DIM))),
        (edge_b2[:, None, :], per_layer((N_LAYERS, 1, DIM))),
        (coor_W1, per_layer((N_LAYERS, DIM, N))),
        (coor_b1[:, None, :], per_layer((N_LAYERS, 1, N))),
        (jnp.swapaxes(coor_W2, 1, 2), per_layer((N_LAYERS, 1, N))),
        (coor_b2[:, :, None], per_layer((N_LAYERS, 1, 1))),
        (ln_g[:, None, :], per_layer((N_LAYERS, 1, DIM))),
        (ln_b[:, None, :], per_layer((N_LAYERS, 1, DIM))),
        (cn_scale[:, :, None], per_layer((N_LAYERS, 1, 1))),
        (node_W1, per_layer((N_LAYERS, 2 * DIM, 2 * DIM))),
        (node_b1[:, None, :], per_layer((N_LAYERS, 1, 2 * DIM))),
        (node_W2, per_layer((N_LAYERS, 2 * DIM, DIM))),
        (node_b2[:, None, :], per_layer((N_LAYERS, 1, DIM))),
    ]

    feats_o, coors_o = pl.pallas_call(
        _egnn_kernel,
        grid=(N_LAYERS,),
        in_specs=[s for _, s in ins],
        out_specs=[
            pl.BlockSpec((N, DIM), lambda l: (0, 0)),
            pl.BlockSpec((N, 128), lambda l: (0, 0)),
        ],
        out_shape=[
            jax.ShapeDtypeStruct((N, DIM), _f32),
            jax.ShapeDtypeStruct((N, 128), _f32),
        ],
        scratch_shapes=[
            pltpu.VMEM((N, DIM), _f32),
            pltpu.VMEM((N, 128), _f32),
            pltpu.VMEM((N, DIM), _f32),
            pltpu.VMEM((N, 128), _f32),
        ],
    )(*[a for a, _ in ins])

    return feats_o.reshape(1, N, DIM), coors_o[:, :3].reshape(1, N, 3)


# speedup vs baseline: 1.0041x; 1.0041x over previous
import functools

import jax
import jax.numpy as jnp
from jax.experimental import pallas as pl
from jax.experimental.pallas import tpu as pltpu

N_LAYERS = 16
DIM = 64
RES_DIM = 62
N = 256
E_HID = (DIM * 2 + 1) * 2
BI = 32
NBLK = N // BI

_f32 = jnp.float32


def _silu(x):
    return x * jax.nn.sigmoid(x)


def _egnn_kernel(
    seq_ref, chain_ref, tab_ref, coords0_ref, mask_ref,
    w1i_ref, w1j_ref, w1d_ref, eb1_ref,
    ew2_ref, eb2_ref,
    cw1_ref, cb1_ref, cw2_ref, cb2_ref,
    lg_ref, lb_ref, cns_ref,
    nw1_ref, nb1_ref, nw2_ref, nb2_ref,
    feats_out_ref, coors_out_ref,
    feats_s, coors_s, mi_s, cnew_s,
):
    l = pl.program_id(0)

    @pl.when(l == 0)
    def _init():
        v = jax.lax.broadcasted_iota(jnp.int32, (N, 24), 1)
        oh = jnp.logical_or(v == seq_ref[...], v == chain_ref[...] + 21)
        feats_s[...] = jnp.dot(oh.astype(_f32), tab_ref[...],
                               preferred_element_type=_f32)
        coors_s[...] = coords0_ref[...]

    feats = feats_s[...]
    C = coors_s[...]

    w1i = w1i_ref[0]
    w1j = w1j_ref[0]
    w1d = w1d_ref[0]
    eb1 = eb1_ref[0]
    ew2 = ew2_ref[0]
    eb2 = eb2_ref[0]
    cw1 = cw1_ref[0]
    cb1 = cb1_ref[0]
    cw2 = cw2_ref[0]
    cb2 = cb2_ref[0][0, 0]
    cns = cns_ref[0][0, 0]

    A = jnp.dot(feats, w1i, preferred_element_type=_f32) + eb1
    B = jnp.dot(feats, w1j, preferred_element_type=_f32)

    CT = jnp.transpose(C[:, :8])
    dx = C[:, 0:1] - CT[0:1, :]
    dy = C[:, 1:2] - CT[1:2, :]
    dz = C[:, 2:3] - CT[2:3, :]
    D = dx * dx + dy * dy + dz * dz
    norm = jnp.sqrt(jnp.maximum(D, 1e-12))
    rinv = cns / jnp.maximum(norm, 1e-8)
    ii = jax.lax.broadcasted_iota(jnp.int32, (N, N), 0)
    jj = jax.lax.broadcasted_iota(jnp.int32, (N, N), 1)
    rinv = jnp.where(ii == jj, 0.0, rinv)

    for ib in range(NBLK):
        i0 = ib * BI
        A_blk = A[i0:i0 + BI, :]
        D_blk = D[i0:i0 + BI, :]
        rinv_blk = rinv[i0:i0 + BI, :]
        C_blk = C[i0:i0 + BI, :]

        D3 = D_blk[:, :, None]
        H = A_blk[:, None, :] + B[None, :, :] + D3 * w1d[None, :, :]
        H2 = H.reshape(BI * N, E_HID)
        m2 = _silu(jnp.dot(_silu(H2), ew2, preferred_element_type=_f32) + eb2)
        u2p = jnp.dot(m2.astype(jnp.bfloat16), cw1,
                      preferred_element_type=_f32) + cb1
        u2 = _silu(u2p.astype(jnp.bfloat16))
        wp = (u2 * cw2.astype(jnp.bfloat16)).astype(_f32)
        w = jnp.sum(wp.reshape(BI, N, N), axis=-1) + cb2
        wc = jnp.clip(w, -2.0, 2.0)
        R = wc * rinv_blk

        mi_s[pl.ds(i0, BI), :] = jnp.sum(m2.reshape(BI, N, DIM), axis=1)
        rs = jnp.sum(R, axis=1, keepdims=True)
        delta = C_blk * rs - jnp.dot(R, C, preferred_element_type=_f32)
        cnew_s[pl.ds(i0, BI), :] = C_blk + delta

    mi = mi_s[...]
    mu = jnp.mean(feats, axis=-1, keepdims=True)
    var = jnp.mean((feats - mu) ** 2, axis=-1, keepdims=True)
    normed = (feats - mu) * jax.lax.rsqrt(var + 1e-5) * lg_ref[0] + lb_ref[0]
    h = jnp.concatenate([normed, mi], axis=-1)
    h = _silu(jnp.dot(h, nw1_ref[0], preferred_element_type=_f32) + nb1_ref[0])
    node = jnp.dot(h, nw2_ref[0], preferred_element_type=_f32) + nb2_ref[0] + feats

    maskf = mask_ref[...]
    cupd = maskf * cnew_s[...] + (1.0 - maskf) * coords0_ref[...]

    feats_s[...] = node
    coors_s[...] = cupd
    feats_out_ref[...] = node
    coors_out_ref[...] = cupd


@jax.jit
def kernel(input_seq, input_chain, input_mask, coords, seq_table, chain_table,
           edge_W1, edge_b1, edge_W2, edge_b2, ln_g, ln_b, cn_scale,
           node_W1, node_b1, node_W2, node_b2, coor_W1, coor_b1, coor_W2,
           coor_b2):
    W1i = edge_W1[:, 0:DIM, :]
    W1j = edge_W1[:, DIM:2 * DIM, :]
    w1d = edge_W1[:, 2 * DIM:2 * DIM + 1, :]
    tab = jnp.zeros((24, DIM), _f32)
    tab = tab.at[0:21, 0:RES_DIM].set(seq_table)
    tab = tab.at[21:23, RES_DIM:DIM].set(chain_table)
    seq2 = input_seq.astype(jnp.int32).reshape(N, 1)
    chain2 = input_chain.astype(jnp.int32).reshape(N, 1)
    coords_pad = jnp.concatenate(
        [coords[0].astype(_f32), jnp.zeros((N, 125), _f32)], axis=1)
    maskf = input_mask.astype(_f32).reshape(N, 1)

    per_layer = lambda s: pl.BlockSpec((1,) + s[1:], lambda l: (l, 0, 0))
    full = lambda s: pl.BlockSpec(s, lambda l: (0, 0))

    ins = [
        (seq2, full((N, 1))),
        (chain2, full((N, 1))),
        (tab, full((24, DIM))),
        (coords_pad, full((N, 128))),
        (maskf, full((N, 1))),
        (W1i, per_layer((N_LAYERS, DIM, E_HID))),
        (W1j, per_layer((N_LAYERS, DIM, E_HID))),
        (w1d, per_layer((N_LAYERS, 1, E_HID))),
        (edge_b1[:, None, :], per_layer((N_LAYERS, 1, E_HID))),
        (edge_W2, per_layer((N_LAYERS, E_HID, DIM))),
        (edge_b2[:, None, :], per_layer((N_LAYERS, 1, DIM))),
        (coor_W1.astype(jnp.bfloat16), per_layer((N_LAYERS, DIM, N))),
        (coor_b1[:, None, :], per_layer((N_LAYERS, 1, N))),
        (jnp.swapaxes(coor_W2, 1, 2), per_layer((N_LAYERS, 1, N))),
        (coor_b2[:, :, None], per_layer((N_LAYERS, 1, 1))),
        (ln_g[:, None, :], per_layer((N_LAYERS, 1, DIM))),
        (ln_b[:, None, :], per_layer((N_LAYERS, 1, DIM))),
        (cn_scale[:, :, None], per_layer((N_LAYERS, 1, 1))),
        (node_W1, per_layer((N_LAYERS, 2 * DIM, 2 * DIM))),
        (node_b1[:, None, :], per_layer((N_LAYERS, 1, 2 * DIM))),
        (node_W2, per_layer((N_LAYERS, 2 * DIM, DIM))),
        (node_b2[:, None, :], per_layer((N_LAYERS, 1, DIM))),
    ]

    feats_o, coors_o = pl.pallas_call(
        _egnn_kernel,
        grid=(N_LAYERS,),
        in_specs=[s for _, s in ins],
        out_specs=[
            pl.BlockSpec((N, DIM), lambda l: (0, 0)),
            pl.BlockSpec((N, 128), lambda l: (0, 0)),
        ],
        out_shape=[
            jax.ShapeDtypeStruct((N, DIM), _f32),
            jax.ShapeDtypeStruct((N, 128), _f32),
        ],
        scratch_shapes=[
            pltpu.VMEM((N, DIM), _f32),
            pltpu.VMEM((N, 128), _f32),
            pltpu.VMEM((N, DIM), _f32),
            pltpu.VMEM((N, 128), _f32),
        ],
    )(*[a for a, _ in ins])

    return feats_o.reshape(1, N, DIM), coors_o[:, :3].reshape(1, N, 3)


# speedup vs baseline: 1.7794x; 1.7722x over previous
import functools

import jax
import jax.numpy as jnp
from jax.experimental import pallas as pl
from jax.experimental.pallas import tpu as pltpu

N_LAYERS = 16
DIM = 64
RES_DIM = 62
N = 256
E_HID = (DIM * 2 + 1) * 2
BI = 32
NBLK = N // BI

_f32 = jnp.float32


def _silu(x):
    return x * jax.nn.sigmoid(x)


def _egnn_kernel(
    seq_ref, chain_ref, tab_ref, coords0_ref, mask_ref,
    w1i_ref, w1j_ref, w1d_ref, eb1_ref,
    ew2_ref, eb2_ref,
    cw1_ref, cb1_ref, cw2_ref, cb2_ref,
    lg_ref, lb_ref, cns_ref,
    nw1_ref, nb1_ref, nw2_ref, nb2_ref,
    feats_out_ref, coors_out_ref,
    feats_s, coors_s, mi_s, cnew_s,
):
    l = pl.program_id(0)

    @pl.when(l == 0)
    def _init():
        v = jax.lax.broadcasted_iota(jnp.int32, (N, 24), 1)
        oh = jnp.logical_or(v == seq_ref[...], v == chain_ref[...] + 21)
        feats_s[...] = jnp.dot(oh.astype(_f32), tab_ref[...],
                               preferred_element_type=_f32)
        coors_s[...] = coords0_ref[...]

    feats = feats_s[...]
    C = coors_s[...]

    w1i = w1i_ref[0]
    w1j = w1j_ref[0]
    w1d = w1d_ref[0]
    eb1 = eb1_ref[0]
    ew2 = ew2_ref[0]
    eb2 = eb2_ref[0]
    cw1 = cw1_ref[0]
    cb1 = cb1_ref[0]
    cw2 = cw2_ref[0]
    cb2 = cb2_ref[0][0, 0]
    cns = cns_ref[0][0, 0]

    A = jnp.dot(feats, w1i, preferred_element_type=_f32) + eb1
    B = jnp.dot(feats, w1j, preferred_element_type=_f32)

    CT = jnp.transpose(C[:, :8])
    dx = C[:, 0:1] - CT[0:1, :]
    dy = C[:, 1:2] - CT[1:2, :]
    dz = C[:, 2:3] - CT[2:3, :]
    D = dx * dx + dy * dy + dz * dz
    norm = jnp.sqrt(jnp.maximum(D, 1e-12))
    rinv = cns / jnp.maximum(norm, 1e-8)
    ii = jax.lax.broadcasted_iota(jnp.int32, (N, N), 0)
    jj = jax.lax.broadcasted_iota(jnp.int32, (N, N), 1)
    rinv = jnp.where(ii == jj, 0.0, rinv)

    for ib in range(NBLK):
        i0 = ib * BI
        A_blk = A[i0:i0 + BI, :]
        D_blk = D[i0:i0 + BI, :]
        rinv_blk = rinv[i0:i0 + BI, :]
        C_blk = C[i0:i0 + BI, :]

        D3 = D_blk[:, :, None]
        Hm = (A_blk[:, None, :256] + B[None, :, :256]
              + D3 * w1d[None, :, :256])
        sm = _silu(Hm).reshape(BI * N, 256)
        Ht = (A_blk[:, None, 256:] + B[None, :, 256:]
              + D3 * w1d[None, :, 256:])
        st = _silu(Ht).reshape(BI * N, 2)
        main = jnp.dot(sm, ew2[:256, :], preferred_element_type=_f32)
        tail = (st[:, 0:1] * ew2[256:257, :] + st[:, 1:2] * ew2[257:258, :])
        m2 = _silu(main + tail + eb2)
        u2p = jnp.dot(m2.astype(jnp.bfloat16), cw1,
                      preferred_element_type=_f32) + cb1
        u2 = _silu(u2p.astype(jnp.bfloat16))
        wp = (u2 * cw2.astype(jnp.bfloat16)).astype(_f32)
        w = jnp.sum(wp.reshape(BI, N, N), axis=-1) + cb2
        wc = jnp.clip(w, -2.0, 2.0)
        R = wc * rinv_blk

        mi_s[pl.ds(i0, BI), :] = jnp.sum(m2.reshape(BI, N, DIM), axis=1)
        rs = jnp.sum(R, axis=1, keepdims=True)
        delta = C_blk * rs - jnp.dot(R, C, preferred_element_type=_f32)
        cnew_s[pl.ds(i0, BI), :] = C_blk + delta

    mi = mi_s[...]
    mu = jnp.mean(feats, axis=-1, keepdims=True)
    var = jnp.mean((feats - mu) ** 2, axis=-1, keepdims=True)
    normed = (feats - mu) * jax.lax.rsqrt(var + 1e-5) * lg_ref[0] + lb_ref[0]
    h = jnp.concatenate([normed, mi], axis=-1)
    h = _silu(jnp.dot(h, nw1_ref[0], preferred_element_type=_f32) + nb1_ref[0])
    node = jnp.dot(h, nw2_ref[0], preferred_element_type=_f32) + nb2_ref[0] + feats

    maskf = mask_ref[...]
    cupd = maskf * cnew_s[...] + (1.0 - maskf) * coords0_ref[...]

    feats_s[...] = node
    coors_s[...] = cupd
    feats_out_ref[...] = node
    coors_out_ref[...] = cupd


@jax.jit
def kernel(input_seq, input_chain, input_mask, coords, seq_table, chain_table,
           edge_W1, edge_b1, edge_W2, edge_b2, ln_g, ln_b, cn_scale,
           node_W1, node_b1, node_W2, node_b2, coor_W1, coor_b1, coor_W2,
           coor_b2):
    W1i = edge_W1[:, 0:DIM, :]
    W1j = edge_W1[:, DIM:2 * DIM, :]
    w1d = edge_W1[:, 2 * DIM:2 * DIM + 1, :]
    tab = jnp.zeros((24, DIM), _f32)
    tab = tab.at[0:21, 0:RES_DIM].set(seq_table)
    tab = tab.at[21:23, RES_DIM:DIM].set(chain_table)
    seq2 = input_seq.astype(jnp.int32).reshape(N, 1)
    chain2 = input_chain.astype(jnp.int32).reshape(N, 1)
    coords_pad = jnp.concatenate(
        [coords[0].astype(_f32), jnp.zeros((N, 125), _f32)], axis=1)
    maskf = input_mask.astype(_f32).reshape(N, 1)

    per_layer = lambda s: pl.BlockSpec((1,) + s[1:], lambda l: (l, 0, 0))
    full = lambda s: pl.BlockSpec(s, lambda l: (0, 0))

    ins = [
        (seq2, full((N, 1))),
        (chain2, full((N, 1))),
        (tab, full((24, DIM))),
        (coords_pad, full((N, 128))),
        (maskf, full((N, 1))),
        (W1i, per_layer((N_LAYERS, DIM, E_HID))),
        (W1j, per_layer((N_LAYERS, DIM, E_HID))),
        (w1d, per_layer((N_LAYERS, 1, E_HID))),
        (edge_b1[:, None, :], per_layer((N_LAYERS, 1, E_HID))),
        (edge_W2, per_layer((N_LAYERS, E_HID, DIM))),
        (edge_b2[:, None, :], per_layer((N_LAYERS, 1, DIM))),
        (coor_W1.astype(jnp.bfloat16), per_layer((N_LAYERS, DIM, N))),
        (coor_b1[:, None, :], per_layer((N_LAYERS, 1, N))),
        (jnp.swapaxes(coor_W2, 1, 2), per_layer((N_LAYERS, 1, N))),
        (coor_b2[:, :, None], per_layer((N_LAYERS, 1, 1))),
        (ln_g[:, None, :], per_layer((N_LAYERS, 1, DIM))),
        (ln_b[:, None, :], per_layer((N_LAYERS, 1, DIM))),
        (cn_scale[:, :, None], per_layer((N_LAYERS, 1, 1))),
        (node_W1, per_layer((N_LAYERS, 2 * DIM, 2 * DIM))),
        (node_b1[:, None, :], per_layer((N_LAYERS, 1, 2 * DIM))),
        (node_W2, per_layer((N_LAYERS, 2 * DIM, DIM))),
        (node_b2[:, None, :], per_layer((N_LAYERS, 1, DIM))),
    ]

    feats_o, coors_o = pl.pallas_call(
        _egnn_kernel,
        grid=(N_LAYERS,),
        in_specs=[s for _, s in ins],
        out_specs=[
            pl.BlockSpec((N, DIM), lambda l: (0, 0)),
            pl.BlockSpec((N, 128), lambda l: (0, 0)),
        ],
        out_shape=[
            jax.ShapeDtypeStruct((N, DIM), _f32),
            jax.ShapeDtypeStruct((N, 128), _f32),
        ],
        scratch_shapes=[
            pltpu.VMEM((N, DIM), _f32),
            pltpu.VMEM((N, 128), _f32),
            pltpu.VMEM((N, DIM), _f32),
            pltpu.VMEM((N, 128), _f32),
        ],
    )(*[a for a, _ in ins])

    return feats_o.reshape(1, N, DIM), coors_o[:, :3].reshape(1, N, 3)


# speedup vs baseline: 2.0510x; 1.1526x over previous
import jax
import jax.numpy as jnp
from jax.experimental import pallas as pl
from jax.experimental.pallas import tpu as pltpu

N_LAYERS = 16
DIM = 64
RES_DIM = 62
N = 256
E_HID = (DIM * 2 + 1) * 2
BI = 32
NBLK = N // BI

_f32 = jnp.float32
_bf16 = jnp.bfloat16


def _silu(x):
    return x * jax.nn.sigmoid(x)


def _egnn_kernel(
    seq_ref, chain_ref, tab_ref, coords0_ref, mask_ref,
    w1i_ref, w1j_ref, w1d_ref, eb1_ref,
    ew2_ref, eb2_ref,
    cw1_ref, cb1_ref, cw2_ref, cb2_ref,
    lg_ref, lb_ref, cns_ref,
    nw1_ref, nb1_ref, nw2_ref, nb2_ref,
    feats_out_ref, coors_out_ref,
    feats_s, coors_s, mi_s, cnew_s, A_s, B_s, D_s, rinv_s,
):
    l = pl.program_id(0)
    ib = pl.program_id(1)

    @pl.when(jnp.logical_and(l == 0, ib == 0))
    def _init():
        v = jax.lax.broadcasted_iota(jnp.int32, (N, 24), 1)
        oh = jnp.logical_or(v == seq_ref[...], v == chain_ref[...] + 21)
        feats_s[...] = jnp.dot(oh.astype(_f32), tab_ref[...],
                               preferred_element_type=_f32)
        coors_s[...] = coords0_ref[...]

    @pl.when(ib == 0)
    def _per_layer():
        feats = feats_s[...]
        C = coors_s[...]
        A_s[...] = jnp.dot(feats, w1i_ref[0],
                           preferred_element_type=_f32) + eb1_ref[0]
        B_s[...] = jnp.dot(feats, w1j_ref[0], preferred_element_type=_f32)
        CT = jnp.transpose(C[:, :8])
        dx = C[:, 0:1] - CT[0:1, :]
        dy = C[:, 1:2] - CT[1:2, :]
        dz = C[:, 2:3] - CT[2:3, :]
        D = dx * dx + dy * dy + dz * dz
        norm = jnp.sqrt(jnp.maximum(D, 1e-12))
        rinv = cns_ref[0][0, 0] / jnp.maximum(norm, 1e-8)
        ii = jax.lax.broadcasted_iota(jnp.int32, (N, N), 0)
        jj = jax.lax.broadcasted_iota(jnp.int32, (N, N), 1)
        D_s[...] = D
        rinv_s[...] = jnp.where(ii == jj, 0.0, rinv)

    i0 = ib * BI
    A_blk = A_s[pl.ds(i0, BI), :]
    B = B_s[...]
    D_blk = D_s[pl.ds(i0, BI), :]
    rinv_blk = rinv_s[pl.ds(i0, BI), :]
    C = coors_s[...]
    C_blk = coors_s[pl.ds(i0, BI), :]
    w1d = w1d_ref[0]
    ew2 = ew2_ref[0]

    D3 = D_blk[:, :, None]
    Hm = (A_blk[:, None, :256] + B[None, :, :256]
          + D3 * w1d[None, :, :256])
    sm = _silu(Hm).reshape(BI * N, 256)
    Ht = (A_blk[:, None, 256:] + B[None, :, 256:]
          + D3 * w1d[None, :, 256:])
    st = _silu(Ht).reshape(BI * N, 2)
    main = jnp.dot(sm, ew2[:256, :], preferred_element_type=_f32)
    tail = st[:, 0:1] * ew2[256:257, :] + st[:, 1:2] * ew2[257:258, :]
    m2 = _silu(main + tail + eb2_ref[0])

    u2p = jnp.dot(m2.astype(_bf16), cw1_ref[0],
                  preferred_element_type=_f32) + cb1_ref[0]
    u2 = _silu(u2p.astype(_bf16))
    wp = (u2 * cw2_ref[0].astype(_bf16)).astype(_f32)
    w = jnp.sum(wp.reshape(BI, N, N), axis=-1) + cb2_ref[0][0, 0]
    wc = jnp.clip(w, -2.0, 2.0)
    R = wc * rinv_blk

    mi_s[pl.ds(i0, BI), :] = jnp.sum(m2.reshape(BI, N, DIM), axis=1)
    rs = jnp.sum(R, axis=1, keepdims=True)
    delta = C_blk * rs - jnp.dot(R, C, preferred_element_type=_f32)
    cnew_s[pl.ds(i0, BI), :] = C_blk + delta

    @pl.when(ib == NBLK - 1)
    def _finish_layer():
        feats = feats_s[...]
        mi = mi_s[...]
        mu = jnp.mean(feats, axis=-1, keepdims=True)
        var = jnp.mean((feats - mu) ** 2, axis=-1, keepdims=True)
        normed = ((feats - mu) * jax.lax.rsqrt(var + 1e-5) * lg_ref[0]
                  + lb_ref[0])
        h = jnp.concatenate([normed, mi], axis=-1)
        h = _silu(jnp.dot(h, nw1_ref[0], preferred_element_type=_f32)
                  + nb1_ref[0])
        node = (jnp.dot(h, nw2_ref[0], preferred_element_type=_f32)
                + nb2_ref[0] + feats)
        maskf = mask_ref[...]
        cupd = maskf * cnew_s[...] + (1.0 - maskf) * coords0_ref[...]
        feats_s[...] = node
        coors_s[...] = cupd
        feats_out_ref[...] = node
        coors_out_ref[...] = cupd


@jax.jit
def kernel(input_seq, input_chain, input_mask, coords, seq_table, chain_table,
           edge_W1, edge_b1, edge_W2, edge_b2, ln_g, ln_b, cn_scale,
           node_W1, node_b1, node_W2, node_b2, coor_W1, coor_b1, coor_W2,
           coor_b2):
    W1i = edge_W1[:, 0:DIM, :]
    W1j = edge_W1[:, DIM:2 * DIM, :]
    w1d = edge_W1[:, 2 * DIM:2 * DIM + 1, :]
    tab = jnp.zeros((24, DIM), _f32)
    tab = tab.at[0:21, 0:RES_DIM].set(seq_table)
    tab = tab.at[21:23, RES_DIM:DIM].set(chain_table)
    seq2 = input_seq.astype(jnp.int32).reshape(N, 1)
    chain2 = input_chain.astype(jnp.int32).reshape(N, 1)
    coords_pad = jnp.concatenate(
        [coords[0].astype(_f32), jnp.zeros((N, 125), _f32)], axis=1)
    maskf = input_mask.astype(_f32).reshape(N, 1)

    per_layer = lambda s: pl.BlockSpec((1,) + s[1:], lambda l, ib: (l, 0, 0))
    full = lambda s: pl.BlockSpec(s, lambda l, ib: (0, 0))

    ins = [
        (seq2, full((N, 1))),
        (chain2, full((N, 1))),
        (tab, full((24, DIM))),
        (coords_pad, full((N, 128))),
        (maskf, full((N, 1))),
        (W1i, per_layer((N_LAYERS, DIM, E_HID))),
        (W1j, per_layer((N_LAYERS, DIM, E_HID))),
        (w1d, per_layer((N_LAYERS, 1, E_HID))),
        (edge_b1[:, None, :], per_layer((N_LAYERS, 1, E_HID))),
        (edge_W2, per_layer((N_LAYERS, E_HID, DIM))),
        (edge_b2[:, None, :], per_layer((N_LAYERS, 1, DIM))),
        (coor_W1.astype(_bf16), per_layer((N_LAYERS, DIM, N))),
        (coor_b1[:, None, :], per_layer((N_LAYERS, 1, N))),
        (jnp.swapaxes(coor_W2, 1, 2), per_layer((N_LAYERS, 1, N))),
        (coor_b2[:, :, None], per_layer((N_LAYERS, 1, 1))),
        (ln_g[:, None, :], per_layer((N_LAYERS, 1, DIM))),
        (ln_b[:, None, :], per_layer((N_LAYERS, 1, DIM))),
        (cn_scale[:, :, None], per_layer((N_LAYERS, 1, 1))),
        (node_W1, per_layer((N_LAYERS, 2 * DIM, 2 * DIM))),
        (node_b1[:, None, :], per_layer((N_LAYERS, 1, 2 * DIM))),
        (node_W2, per_layer((N_LAYERS, 2 * DIM, DIM))),
        (node_b2[:, None, :], per_layer((N_LAYERS, 1, DIM))),
    ]

    feats_o, coors_o = pl.pallas_call(
        _egnn_kernel,
        grid=(N_LAYERS, NBLK),
        in_specs=[s for _, s in ins],
        out_specs=[
            pl.BlockSpec((N, DIM), lambda l, ib: (0, 0)),
            pl.BlockSpec((N, 128), lambda l, ib: (0, 0)),
        ],
        out_shape=[
            jax.ShapeDtypeStruct((N, DIM), _f32),
            jax.ShapeDtypeStruct((N, 128), _f32),
        ],
        scratch_shapes=[
            pltpu.VMEM((N, DIM), _f32),
            pltpu.VMEM((N, 128), _f32),
            pltpu.VMEM((N, DIM), _f32),
            pltpu.VMEM((N, 128), _f32),
            pltpu.VMEM((N, E_HID), _f32),
            pltpu.VMEM((N, E_HID), _f32),
            pltpu.VMEM((N, N), _f32),
            pltpu.VMEM((N, N), _f32),
        ],
    )(*[a for a, _ in ins])

    return feats_o.reshape(1, N, DIM), coors_o[:, :3].reshape(1, N, 3)


# speedup vs baseline: 2.0689x; 1.0088x over previous
import jax
import jax.numpy as jnp
from jax.experimental import pallas as pl
from jax.experimental.pallas import tpu as pltpu

N_LAYERS = 16
DIM = 64
RES_DIM = 62
N = 256
E_HID = (DIM * 2 + 1) * 2
BI = 64
NBLK = N // BI

_f32 = jnp.float32
_bf16 = jnp.bfloat16


def _silu(x):
    return x * jax.nn.sigmoid(x)


def _egnn_kernel(
    seq_ref, chain_ref, tab_ref, coords0_ref, mask_ref,
    w1i_ref, w1j_ref, w1d_ref, eb1_ref,
    ew2_ref, eb2_ref,
    cw1_ref, cb1_ref, cw2_ref, cb2_ref,
    lg_ref, lb_ref, cns_ref,
    nw1_ref, nb1_ref, nw2_ref, nb2_ref,
    feats_out_ref, coors_out_ref,
    feats_s, coors_s, mi_s, cnew_s, A_s, B_s, D_s, rinv_s,
):
    l = pl.program_id(0)
    ib = pl.program_id(1)

    @pl.when(jnp.logical_and(l == 0, ib == 0))
    def _init():
        v = jax.lax.broadcasted_iota(jnp.int32, (N, 24), 1)
        oh = jnp.logical_or(v == seq_ref[...], v == chain_ref[...] + 21)
        feats_s[...] = jnp.dot(oh.astype(_f32), tab_ref[...],
                               preferred_element_type=_f32)
        coors_s[...] = coords0_ref[...]

    @pl.when(ib == 0)
    def _per_layer():
        feats = feats_s[...]
        C = coors_s[...]
        A_s[...] = jnp.dot(feats, w1i_ref[0],
                           preferred_element_type=_f32) + eb1_ref[0]
        B_s[...] = jnp.dot(feats, w1j_ref[0], preferred_element_type=_f32)
        CT = jnp.transpose(C[:, :8])
        dx = C[:, 0:1] - CT[0:1, :]
        dy = C[:, 1:2] - CT[1:2, :]
        dz = C[:, 2:3] - CT[2:3, :]
        D = dx * dx + dy * dy + dz * dz
        norm = jnp.sqrt(jnp.maximum(D, 1e-12))
        rinv = cns_ref[0][0, 0] / jnp.maximum(norm, 1e-8)
        ii = jax.lax.broadcasted_iota(jnp.int32, (N, N), 0)
        jj = jax.lax.broadcasted_iota(jnp.int32, (N, N), 1)
        D_s[...] = D
        rinv_s[...] = jnp.where(ii == jj, 0.0, rinv)

    i0 = ib * BI
    A_blk = A_s[pl.ds(i0, BI), :]
    B = B_s[...]
    D_blk = D_s[pl.ds(i0, BI), :]
    rinv_blk = rinv_s[pl.ds(i0, BI), :]
    C = coors_s[...]
    C_blk = coors_s[pl.ds(i0, BI), :]
    w1d = w1d_ref[0]
    ew2 = ew2_ref[0]

    D3 = D_blk[:, :, None]
    Hm = (A_blk[:, None, :256] + B[None, :, :256]
          + D3 * w1d[None, :, :256])
    sm = _silu(Hm).reshape(BI * N, 256)
    Ht = (A_blk[:, None, 256:] + B[None, :, 256:]
          + D3 * w1d[None, :, 256:])
    st = _silu(Ht).reshape(BI * N, 2)
    main = jnp.dot(sm, ew2[:256, :], preferred_element_type=_f32)
    tail = st[:, 0:1] * ew2[256:257, :] + st[:, 1:2] * ew2[257:258, :]
    m2 = _silu(main + tail + eb2_ref[0])

    u2p = jnp.dot(m2.astype(_bf16), cw1_ref[0],
                  preferred_element_type=_f32) + cb1_ref[0]
    u2 = _silu(u2p.astype(_bf16))
    wp = (u2 * cw2_ref[0].astype(_bf16)).astype(_f32)
    w = jnp.sum(wp.reshape(BI, N, N), axis=-1) + cb2_ref[0][0, 0]
    wc = jnp.clip(w, -2.0, 2.0)
    R = wc * rinv_blk

    mi_s[pl.ds(i0, BI), :] = jnp.sum(m2.reshape(BI, N, DIM), axis=1)
    rs = jnp.sum(R, axis=1, keepdims=True)
    delta = C_blk * rs - jnp.dot(R, C, preferred_element_type=_f32)
    cnew_s[pl.ds(i0, BI), :] = C_blk + delta

    @pl.when(ib == NBLK - 1)
    def _finish_layer():
        feats = feats_s[...]
        mi = mi_s[...]
        mu = jnp.mean(feats, axis=-1, keepdims=True)
        var = jnp.mean((feats - mu) ** 2, axis=-1, keepdims=True)
        normed = ((feats - mu) * jax.lax.rsqrt(var + 1e-5) * lg_ref[0]
                  + lb_ref[0])
        h = jnp.concatenate([normed, mi], axis=-1)
        h = _silu(jnp.dot(h, nw1_ref[0], preferred_element_type=_f32)
                  + nb1_ref[0])
        node = (jnp.dot(h, nw2_ref[0], preferred_element_type=_f32)
                + nb2_ref[0] + feats)
        maskf = mask_ref[...]
        cupd = maskf * cnew_s[...] + (1.0 - maskf) * coords0_ref[...]
        feats_s[...] = node
        coors_s[...] = cupd
        feats_out_ref[...] = node
        coors_out_ref[...] = cupd


@jax.jit
def kernel(input_seq, input_chain, input_mask, coords, seq_table, chain_table,
           edge_W1, edge_b1, edge_W2, edge_b2, ln_g, ln_b, cn_scale,
           node_W1, node_b1, node_W2, node_b2, coor_W1, coor_b1, coor_W2,
           coor_b2):
    W1i = edge_W1[:, 0:DIM, :]
    W1j = edge_W1[:, DIM:2 * DIM, :]
    w1d = edge_W1[:, 2 * DIM:2 * DIM + 1, :]
    tab = jnp.zeros((24, DIM), _f32)
    tab = tab.at[0:21, 0:RES_DIM].set(seq_table)
    tab = tab.at[21:23, RES_DIM:DIM].set(chain_table)
    seq2 = input_seq.astype(jnp.int32).reshape(N, 1)
    chain2 = input_chain.astype(jnp.int32).reshape(N, 1)
    coords_pad = jnp.concatenate(
        [coords[0].astype(_f32), jnp.zeros((N, 125), _f32)], axis=1)
    maskf = input_mask.astype(_f32).reshape(N, 1)

    per_layer = lambda s: pl.BlockSpec((1,) + s[1:], lambda l, ib: (l, 0, 0))
    full = lambda s: pl.BlockSpec(s, lambda l, ib: (0, 0))

    ins = [
        (seq2, full((N, 1))),
        (chain2, full((N, 1))),
        (tab, full((24, DIM))),
        (coords_pad, full((N, 128))),
        (maskf, full((N, 1))),
        (W1i, per_layer((N_LAYERS, DIM, E_HID))),
        (W1j, per_layer((N_LAYERS, DIM, E_HID))),
        (w1d, per_layer((N_LAYERS, 1, E_HID))),
        (edge_b1[:, None, :], per_layer((N_LAYERS, 1, E_HID))),
        (edge_W2, per_layer((N_LAYERS, E_HID, DIM))),
        (edge_b2[:, None, :], per_layer((N_LAYERS, 1, DIM))),
        (coor_W1.astype(_bf16), per_layer((N_LAYERS, DIM, N))),
        (coor_b1[:, None, :], per_layer((N_LAYERS, 1, N))),
        (jnp.swapaxes(coor_W2, 1, 2), per_layer((N_LAYERS, 1, N))),
        (coor_b2[:, :, None], per_layer((N_LAYERS, 1, 1))),
        (ln_g[:, None, :], per_layer((N_LAYERS, 1, DIM))),
        (ln_b[:, None, :], per_layer((N_LAYERS, 1, DIM))),
        (cn_scale[:, :, None], per_layer((N_LAYERS, 1, 1))),
        (node_W1, per_layer((N_LAYERS, 2 * DIM, 2 * DIM))),
        (node_b1[:, None, :], per_layer((N_LAYERS, 1, 2 * DIM))),
        (node_W2, per_layer((N_LAYERS, 2 * DIM, DIM))),
        (node_b2[:, None, :], per_layer((N_LAYERS, 1, DIM))),
    ]

    feats_o, coors_o = pl.pallas_call(
        _egnn_kernel,
        grid=(N_LAYERS, NBLK),
        in_specs=[s for _, s in ins],
        out_specs=[
            pl.BlockSpec((N, DIM), lambda l, ib: (0, 0)),
            pl.BlockSpec((N, 128), lambda l, ib: (0, 0)),
        ],
        out_shape=[
            jax.ShapeDtypeStruct((N, DIM), _f32),
            jax.ShapeDtypeStruct((N, 128), _f32),
        ],
        scratch_shapes=[
            pltpu.VMEM((N, DIM), _f32),
            pltpu.VMEM((N, 128), _f32),
            pltpu.VMEM((N, DIM), _f32),
            pltpu.VMEM((N, 128), _f32),
            pltpu.VMEM((N, E_HID), _f32),
            pltpu.VMEM((N, E_HID), _f32),
            pltpu.VMEM((N, N), _f32),
            pltpu.VMEM((N, N), _f32),
        ],
    )(*[a for a, _ in ins])

    return feats_o.reshape(1, N, DIM), coors_o[:, :3].reshape(1, N, 3)


# speedup vs baseline: 2.0800x; 1.0053x over previous
import jax
import jax.numpy as jnp
from jax.experimental import pallas as pl
from jax.experimental.pallas import tpu as pltpu

N_LAYERS = 16
DIM = 64
RES_DIM = 62
N = 256
E_HID = (DIM * 2 + 1) * 2
BI = 64
NBLK = N // BI

_f32 = jnp.float32
_bf16 = jnp.bfloat16


def _silu(x):
    return x * jax.nn.sigmoid(x)


def _egnn_kernel(
    seq_ref, chain_ref, tab_ref, coords0_ref, mask_ref,
    w1i_ref, w1j_ref, w1d_ref, eb1_ref,
    ew2_ref, eb2_ref,
    cw1_ref, cb1_ref, cw2_ref, cb2_ref,
    lg_ref, lb_ref, cns_ref,
    nw1_ref, nb1_ref, nw2_ref, nb2_ref,
    feats_out_ref, coors_out_ref,
    feats_s, coors_s, mi_s, cnew_s, A_s, B_s, D_s, rinv_s,
):
    l = pl.program_id(0)
    ib = pl.program_id(1)

    @pl.when(jnp.logical_and(l == 0, ib == 0))
    def _init():
        v = jax.lax.broadcasted_iota(jnp.int32, (N, 24), 1)
        oh = jnp.logical_or(v == seq_ref[...], v == chain_ref[...] + 21)
        feats_s[...] = jnp.dot(oh.astype(_f32), tab_ref[...],
                               preferred_element_type=_f32)
        coors_s[...] = coords0_ref[...]

    @pl.when(ib == 0)
    def _per_layer():
        feats = feats_s[...]
        C = coors_s[...]
        A_s[...] = jnp.dot(feats, w1i_ref[0],
                           preferred_element_type=_f32) + eb1_ref[0]
        B_s[...] = jnp.dot(feats, w1j_ref[0], preferred_element_type=_f32)
        CT = jnp.transpose(C[:, :8])
        dx = C[:, 0:1] - CT[0:1, :]
        dy = C[:, 1:2] - CT[1:2, :]
        dz = C[:, 2:3] - CT[2:3, :]
        D = dx * dx + dy * dy + dz * dz
        norm = jnp.sqrt(jnp.maximum(D, 1e-12))
        rinv = cns_ref[0][0, 0] / jnp.maximum(norm, 1e-8)
        ii = jax.lax.broadcasted_iota(jnp.int32, (N, N), 0)
        jj = jax.lax.broadcasted_iota(jnp.int32, (N, N), 1)
        D_s[...] = D
        rinv_s[...] = jnp.where(ii == jj, 0.0, rinv)

    i0 = ib * BI
    A_blk = A_s[pl.ds(i0, BI), :]
    B = B_s[...]
    D_blk = D_s[pl.ds(i0, BI), :]
    rinv_blk = rinv_s[pl.ds(i0, BI), :]
    C = coors_s[...]
    C_blk = coors_s[pl.ds(i0, BI), :]
    w1d = w1d_ref[0]
    ew2 = ew2_ref[0]

    D3 = D_blk[:, :, None]
    Hm = (A_blk[:, None, :256] + B[None, :, :256]
          + D3 * w1d[None, :, :256])
    sm = _silu(Hm).reshape(BI * N, 256)
    Ht = (A_blk[:, None, 256:] + B[None, :, 256:]
          + D3 * w1d[None, :, 256:])
    st = _silu(Ht).reshape(BI * N, 2)
    main = jnp.dot(sm, ew2[:256, :], preferred_element_type=_f32)
    tail = st[:, 0:1] * ew2[256:257, :] + st[:, 1:2] * ew2[257:258, :]
    m2 = _silu(main + tail + eb2_ref[0])

    u2p = jnp.dot(m2, cw1_ref[0], preferred_element_type=_f32) + cb1_ref[0]
    u2 = _silu(u2p)
    wp = u2 * cw2_ref[0]
    w = jnp.sum(wp.reshape(BI, N, N), axis=-1) + cb2_ref[0][0, 0]
    wc = jnp.clip(w, -2.0, 2.0)
    R = wc * rinv_blk

    mi_s[pl.ds(i0, BI), :] = jnp.sum(m2.reshape(BI, N, DIM), axis=1)
    rs = jnp.sum(R, axis=1, keepdims=True)
    delta = C_blk * rs - jnp.dot(R, C, preferred_element_type=_f32)
    cnew_s[pl.ds(i0, BI), :] = C_blk + delta

    @pl.when(ib == NBLK - 1)
    def _finish_layer():
        feats = feats_s[...]
        mi = mi_s[...]
        mu = jnp.mean(feats, axis=-1, keepdims=True)
        var = jnp.mean((feats - mu) ** 2, axis=-1, keepdims=True)
        normed = ((feats - mu) * jax.lax.rsqrt(var + 1e-5) * lg_ref[0]
                  + lb_ref[0])
        h = jnp.concatenate([normed, mi], axis=-1)
        h = _silu(jnp.dot(h, nw1_ref[0], preferred_element_type=_f32)
                  + nb1_ref[0])
        node = (jnp.dot(h, nw2_ref[0], preferred_element_type=_f32)
                + nb2_ref[0] + feats)
        maskf = mask_ref[...]
        cupd = maskf * cnew_s[...] + (1.0 - maskf) * coords0_ref[...]
        feats_s[...] = node
        coors_s[...] = cupd
        feats_out_ref[...] = node
        coors_out_ref[...] = cupd


@jax.jit
def kernel(input_seq, input_chain, input_mask, coords, seq_table, chain_table,
           edge_W1, edge_b1, edge_W2, edge_b2, ln_g, ln_b, cn_scale,
           node_W1, node_b1, node_W2, node_b2, coor_W1, coor_b1, coor_W2,
           coor_b2):
    W1i = edge_W1[:, 0:DIM, :]
    W1j = edge_W1[:, DIM:2 * DIM, :]
    w1d = edge_W1[:, 2 * DIM:2 * DIM + 1, :]
    tab = jnp.zeros((24, DIM), _f32)
    tab = tab.at[0:21, 0:RES_DIM].set(seq_table)
    tab = tab.at[21:23, RES_DIM:DIM].set(chain_table)
    seq2 = input_seq.astype(jnp.int32).reshape(N, 1)
    chain2 = input_chain.astype(jnp.int32).reshape(N, 1)
    coords_pad = jnp.concatenate(
        [coords[0].astype(_f32), jnp.zeros((N, 125), _f32)], axis=1)
    maskf = input_mask.astype(_f32).reshape(N, 1)

    per_layer = lambda s: pl.BlockSpec((1,) + s[1:], lambda l, ib: (l, 0, 0))
    full = lambda s: pl.BlockSpec(s, lambda l, ib: (0, 0))

    ins = [
        (seq2, full((N, 1))),
        (chain2, full((N, 1))),
        (tab, full((24, DIM))),
        (coords_pad, full((N, 128))),
        (maskf, full((N, 1))),
        (W1i, per_layer((N_LAYERS, DIM, E_HID))),
        (W1j, per_layer((N_LAYERS, DIM, E_HID))),
        (w1d, per_layer((N_LAYERS, 1, E_HID))),
        (edge_b1[:, None, :], per_layer((N_LAYERS, 1, E_HID))),
        (edge_W2, per_layer((N_LAYERS, E_HID, DIM))),
        (edge_b2[:, None, :], per_layer((N_LAYERS, 1, DIM))),
        (coor_W1, per_layer((N_LAYERS, DIM, N))),
        (coor_b1[:, None, :], per_layer((N_LAYERS, 1, N))),
        (jnp.swapaxes(coor_W2, 1, 2), per_layer((N_LAYERS, 1, N))),
        (coor_b2[:, :, None], per_layer((N_LAYERS, 1, 1))),
        (ln_g[:, None, :], per_layer((N_LAYERS, 1, DIM))),
        (ln_b[:, None, :], per_layer((N_LAYERS, 1, DIM))),
        (cn_scale[:, :, None], per_layer((N_LAYERS, 1, 1))),
        (node_W1, per_layer((N_LAYERS, 2 * DIM, 2 * DIM))),
        (node_b1[:, None, :], per_layer((N_LAYERS, 1, 2 * DIM))),
        (node_W2, per_layer((N_LAYERS, 2 * DIM, DIM))),
        (node_b2[:, None, :], per_layer((N_LAYERS, 1, DIM))),
    ]

    feats_o, coors_o = pl.pallas_call(
        _egnn_kernel,
        grid=(N_LAYERS, NBLK),
        in_specs=[s for _, s in ins],
        out_specs=[
            pl.BlockSpec((N, DIM), lambda l, ib: (0, 0)),
            pl.BlockSpec((N, 128), lambda l, ib: (0, 0)),
        ],
        out_shape=[
            jax.ShapeDtypeStruct((N, DIM), _f32),
            jax.ShapeDtypeStruct((N, 128), _f32),
        ],
        scratch_shapes=[
            pltpu.VMEM((N, DIM), _f32),
            pltpu.VMEM((N, 128), _f32),
            pltpu.VMEM((N, DIM), _f32),
            pltpu.VMEM((N, 128), _f32),
            pltpu.VMEM((N, E_HID), _f32),
            pltpu.VMEM((N, E_HID), _f32),
            pltpu.VMEM((N, N), _f32),
            pltpu.VMEM((N, N), _f32),
        ],
    )(*[a for a, _ in ins])

    return feats_o.reshape(1, N, DIM), coors_o[:, :3].reshape(1, N, 3)
